# baseline (speedup 1.0000x reference)
import jax
import jax.numpy as jnp
from jax import lax
from jax.experimental import pallas as pl
from jax.experimental.pallas import tpu as pltpu
from jax.experimental.pallas import tpu_sc as plsc

NBINS = 256
NROWS = 24
NCOLS = 262144
TOTAL = NROWS * NCOLS

NC = 2
NS = 16
NW = NC * NS
SPAN = TOTAL // NW
CHUNK = 16384
CPT = SPAN // CHUNK
ROWS_PER_CHUNK_SHIFT = 4
UNROLL = 8
HIST_WORDS = NROWS * NBINS


def _minmax_body(x_ref, mn_ref, mx_ref):
    i = pl.program_id(0)

    @pl.when(i == 0)
    def _():
        mn_ref[...] = jnp.full((1, 16), jnp.inf, jnp.float32)
        mx_ref[...] = jnp.full((1, 16), -jnp.inf, jnp.float32)

    blk = x_ref[...]
    mn_ref[...] = jnp.minimum(mn_ref[...], jnp.min(blk))
    mx_ref[...] = jnp.maximum(mx_ref[...], jnp.max(blk))


def _minmax(x):
    b, c, h, w = x.shape
    return pl.pallas_call(
        _minmax_body,
        grid=(b,),
        in_specs=[pl.BlockSpec((1, c, h, w), lambda i: (i, 0, 0, 0))],
        out_specs=[
            pl.BlockSpec((1, 16), lambda i: (0, 0)),
            pl.BlockSpec((1, 16), lambda i: (0, 0)),
        ],
        out_shape=[
            jax.ShapeDtypeStruct((1, 16), jnp.float32),
            jax.ShapeDtypeStruct((1, 16), jnp.float32),
        ],
    )(x)


CHUNK_ROWS = CHUNK // 512


def _hist_body(x_hbm, mn_hbm, mx_hbm, out_hbm, xb0, xb1, hist, mnb, mxb, s0, s1):
    wid = lax.axis_index("s") * NC + lax.axis_index("c")
    pltpu.sync_copy(mn_hbm, mnb)
    pltpu.sync_copy(mx_hbm, mxb)
    mnv = mnb[...]
    scale = mxb[...] - mnv

    def zero_body(i, _):
        base = i * 128
        for u in range(8):
            hist[pl.ds(base + u * 16, 16)] = jnp.zeros((16,), jnp.float32)
        return 0

    lax.fori_loop(0, 2 * HIST_WORDS // 128, zero_body, 0)

    ones = jnp.ones((16,), jnp.float32)
    bufs = (xb0, xb1)
    sems = (s0, s1)
    handles = [None, None]

    def start_chunk(c, slot):
        g = wid * CPT + c
        row = g >> ROWS_PER_CHUNK_SHIFT
        r0 = (g & (NCOLS // CHUNK - 1)) * CHUNK_ROWS
        return pltpu.async_copy(
            x_hbm.at[row, pl.ds(r0, CHUNK_ROWS), :], bufs[slot], sems[slot]
        )

    handles[0] = start_chunk(0, 0)
    for c in range(CPT):
        nxt = c + 1
        if nxt < CPT:
            handles[nxt % 2] = start_chunk(nxt, nxt % 2)
        handles[c % 2].wait()
        xb = bufs[c % 2]
        rowoff = ((wid * CPT + c) >> ROWS_PER_CHUNK_SHIFT) * NBINS

        @plsc.parallel_loop(0, CHUNK // 16, 2, unroll=UNROLL // 2)
        def _(j):
            for par, hoff in ((0, 0), (1, HIST_WORDS)):
                jj = j + par
                xv = xb[jj >> 5, pl.ds((jj & 31) * 16, 16)]
                t = (xv - mnv) / scale * 256.0
                t = jnp.minimum(t, 255.0)
                plsc.addupdate_scatter(
                    hist, [t.astype(jnp.int32) + (rowoff + hoff)], ones
                )

    def merge_body(i, _):
        base = i * 128
        for u in range(8):
            o = base + u * 16
            hist[pl.ds(o, 16)] = hist[pl.ds(o, 16)] + hist[pl.ds(HIST_WORDS + o, 16)]
        return 0

    lax.fori_loop(0, HIST_WORDS // 128, merge_body, 0)
    pltpu.sync_copy(
        hist.at[pl.ds(0, HIST_WORDS)],
        out_hbm.at[pl.ds(wid * HIST_WORDS, HIST_WORDS)],
    )


def _hist_partials(x3, mn16, mx16):
    mesh = plsc.VectorSubcoreMesh(core_axis_name="c", subcore_axis_name="s")
    kern = pl.kernel(
        _hist_body,
        out_type=jax.ShapeDtypeStruct((NW * HIST_WORDS,), jnp.float32),
        mesh=mesh,
        compiler_params=pltpu.CompilerParams(
            needs_layout_passes=False, use_tc_tiling_on_sc=True
        ),
        scratch_types=[
            pltpu.VMEM((CHUNK_ROWS, 512), jnp.float32),
            pltpu.VMEM((CHUNK_ROWS, 512), jnp.float32),
            pltpu.VMEM((2 * HIST_WORDS,), jnp.float32),
            pltpu.VMEM((16,), jnp.float32),
            pltpu.VMEM((16,), jnp.float32),
            pltpu.SemaphoreType.DMA,
            pltpu.SemaphoreType.DMA,
        ],
    )
    return kern(x3, mn16, mx16)


def _cumsum_lanes(a):
    n = a.shape[-1]
    k = 1
    while k < n:
        shifted = jnp.concatenate(
            [jnp.zeros(a.shape[:-1] + (k,), a.dtype), a[..., : n - k]], axis=-1
        )
        a = a + shifted
        k *= 2
    return a


def _scan_body(p_ref, mn_ref, mx_ref, thr_ref):
    acc = p_ref[pl.ds(0, HIST_WORDS)]
    for c in range(1, NW):
        acc = acc + p_ref[pl.ds(c * HIST_WORDS, HIST_WORDS)]
    cnt = acc.reshape(NROWS, NBINS)
    ci = cnt.astype(jnp.int32)
    t = lax.broadcasted_iota(jnp.int32, (NROWS, NBINS), 1)
    cc = _cumsum_lanes(ci)
    cct = _cumsum_lanes(ci * t)
    total = cc[:, NBINS - 1 :].astype(jnp.float32)
    omega = cc.astype(jnp.float32) / total
    mu = cct.astype(jnp.float32) / total
    mu_total = mu[:, NBINS - 1 :]
    weight_fg = 1.0 - omega
    valid = (omega > 0.0) & (weight_fg > 0.0)
    mean_bg = mu / jnp.where(omega > 0.0, omega, 1.0)
    mean_fg = (mu_total - mu) / jnp.where(weight_fg > 0.0, weight_fg, 1.0)
    icv = omega * weight_fg * (mean_bg - mean_fg) ** 2
    icv = jnp.where(valid, icv, -jnp.inf)
    m = jnp.max(icv, axis=1, keepdims=True)
    cand = jnp.where(icv == m, t, NBINS)
    best = jnp.min(cand, axis=1, keepdims=True)
    tidx = jnp.clip(best + 1, 0, NBINS - 1)
    mn = mn_ref[0, 0]
    mx = mx_ref[0, 0]
    s = tidx.astype(jnp.float32) / float(NBINS - 1)
    thr = mn * (1.0 - s) + mx * s
    thr_ref[...] = jnp.where(tidx == NBINS - 1, mx, thr)


def _thresholds(partials, mn16, mx16):
    return pl.pallas_call(
        _scan_body,
        out_shape=jax.ShapeDtypeStruct((NROWS, 1), jnp.float32),
    )(partials, mn16, mx16)


def _mask_body(x_ref, t_ref, o_ref):
    row = pl.program_id(0) * 3 + pl.program_id(1)
    rid = lax.broadcasted_iota(jnp.int32, (NROWS, 1), 0)
    tval = jnp.sum(jnp.where(rid == row, t_ref[...], 0.0))
    x = x_ref[...]
    o_ref[...] = jnp.where(x <= tval, 0.0, x)


def _mask(x, thr):
    b, c, h, w = x.shape
    return pl.pallas_call(
        _mask_body,
        grid=(b, c),
        in_specs=[
            pl.BlockSpec((1, 1, h, w), lambda i, j: (i, j, 0, 0)),
            pl.BlockSpec((NROWS, 1), lambda i, j: (0, 0)),
        ],
        out_specs=pl.BlockSpec((1, 1, h, w), lambda i, j: (i, j, 0, 0)),
        out_shape=jax.ShapeDtypeStruct(x.shape, jnp.float32),
    )(x, thr)


@jax.jit
def _run(x):
    mn16, mx16 = _minmax(x)
    partials = _hist_partials(
        x.reshape(NROWS, 512, 512), mn16.reshape(16), mx16.reshape(16)
    )
    thr = _thresholds(partials, mn16, mx16)
    out = _mask(x, thr)
    return out, thr.reshape(NROWS)


def kernel(x, nbins):
    return _run(x)


# speedup vs baseline: 1.0380x; 1.0380x over previous
import jax
import jax.numpy as jnp
from jax import lax
from jax.experimental import pallas as pl
from jax.experimental.pallas import tpu as pltpu
from jax.experimental.pallas import tpu_sc as plsc

NBINS = 256
NROWS = 24
NCOLS = 262144
TOTAL = NROWS * NCOLS

NC = 2
NS = 16
NW = NC * NS
SPAN = TOTAL // NW
CHUNK = 16384
CPT = SPAN // CHUNK
ROWS_PER_CHUNK_SHIFT = 4
UNROLL = 8
HIST_WORDS = NROWS * NBINS


def _minmax_body(x_ref, mn_ref, mx_ref):
    i = pl.program_id(0)

    @pl.when(i == 0)
    def _():
        mn_ref[...] = jnp.full((1, 16), jnp.inf, jnp.float32)
        mx_ref[...] = jnp.full((1, 16), -jnp.inf, jnp.float32)

    blk = x_ref[...]
    mn_ref[...] = jnp.minimum(mn_ref[...], jnp.min(blk))
    mx_ref[...] = jnp.maximum(mx_ref[...], jnp.max(blk))


def _minmax(x):
    b, c, h, w = x.shape
    return pl.pallas_call(
        _minmax_body,
        grid=(b,),
        in_specs=[pl.BlockSpec((1, c, h, w), lambda i: (i, 0, 0, 0))],
        out_specs=[
            pl.BlockSpec((1, 16), lambda i: (0, 0)),
            pl.BlockSpec((1, 16), lambda i: (0, 0)),
        ],
        out_shape=[
            jax.ShapeDtypeStruct((1, 16), jnp.float32),
            jax.ShapeDtypeStruct((1, 16), jnp.float32),
        ],
    )(x)


CHUNK_ROWS = CHUNK // 512


def _hist_body(x_hbm, mn_hbm, mx_hbm, out_hbm, xb0, xb1, hist, mnb, mxb, s0, s1):
    wid = lax.axis_index("s") * NC + lax.axis_index("c")
    pltpu.sync_copy(mn_hbm, mnb)
    pltpu.sync_copy(mx_hbm, mxb)
    mnv = mnb[...]
    scale = mxb[...] - mnv

    def zero_body(i, _):
        base = i * 128
        for u in range(8):
            hist[pl.ds(base + u * 16, 16)] = jnp.zeros((16,), jnp.float32)
        return 0

    lax.fori_loop(0, HIST_WORDS // 128, zero_body, 0)

    ones = jnp.ones((16,), jnp.float32)
    bufs = (xb0, xb1)
    sems = (s0, s1)
    handles = [None, None]

    def start_chunk(c, slot):
        g = wid * CPT + c
        row = g >> ROWS_PER_CHUNK_SHIFT
        r0 = (g & (NCOLS // CHUNK - 1)) * CHUNK_ROWS
        return pltpu.async_copy(
            x_hbm.at[row, pl.ds(r0, CHUNK_ROWS), :], bufs[slot], sems[slot]
        )

    handles[0] = start_chunk(0, 0)
    for c in range(CPT):
        nxt = c + 1
        if nxt < CPT:
            handles[nxt % 2] = start_chunk(nxt, nxt % 2)
        handles[c % 2].wait()
        xb = bufs[c % 2]
        rowoff = ((wid * CPT + c) >> ROWS_PER_CHUNK_SHIFT) * NBINS

        @plsc.parallel_loop(0, CHUNK // 16, 1, unroll=UNROLL)
        def _(j):
            xv = xb[j >> 5, pl.ds((j & 31) * 16, 16)]
            t = (xv - mnv) / scale * 256.0
            t = jnp.minimum(t, 255.0)
            plsc.addupdate_scatter(hist, [t.astype(jnp.int32) + rowoff], ones)

    pltpu.sync_copy(hist, out_hbm.at[pl.ds(wid * HIST_WORDS, HIST_WORDS)])


def _hist_partials(x3, mn16, mx16):
    mesh = plsc.VectorSubcoreMesh(core_axis_name="c", subcore_axis_name="s")
    kern = pl.kernel(
        _hist_body,
        out_type=jax.ShapeDtypeStruct((NW * HIST_WORDS,), jnp.float32),
        mesh=mesh,
        compiler_params=pltpu.CompilerParams(
            needs_layout_passes=False, use_tc_tiling_on_sc=True
        ),
        scratch_types=[
            pltpu.VMEM((CHUNK_ROWS, 512), jnp.float32),
            pltpu.VMEM((CHUNK_ROWS, 512), jnp.float32),
            pltpu.VMEM((HIST_WORDS,), jnp.float32),
            pltpu.VMEM((16,), jnp.float32),
            pltpu.VMEM((16,), jnp.float32),
            pltpu.SemaphoreType.DMA,
            pltpu.SemaphoreType.DMA,
        ],
    )
    return kern(x3, mn16, mx16)


def _cumsum_lanes(a):
    n = a.shape[-1]
    k = 1
    while k < n:
        shifted = jnp.concatenate(
            [jnp.zeros(a.shape[:-1] + (k,), a.dtype), a[..., : n - k]], axis=-1
        )
        a = a + shifted
        k *= 2
    return a


def _compute_thresholds(p_ref, mn_ref, mx_ref):
    acc = p_ref[pl.ds(0, HIST_WORDS)]
    for c in range(1, NW):
        acc = acc + p_ref[pl.ds(c * HIST_WORDS, HIST_WORDS)]
    cnt = acc.reshape(NROWS, NBINS)
    ci = cnt.astype(jnp.int32)
    t = lax.broadcasted_iota(jnp.int32, (NROWS, NBINS), 1)
    cc = _cumsum_lanes(ci)
    cct = _cumsum_lanes(ci * t)
    total = cc[:, NBINS - 1 :].astype(jnp.float32)
    omega = cc.astype(jnp.float32) / total
    mu = cct.astype(jnp.float32) / total
    mu_total = mu[:, NBINS - 1 :]
    weight_fg = 1.0 - omega
    valid = (omega > 0.0) & (weight_fg > 0.0)
    mean_bg = mu / jnp.where(omega > 0.0, omega, 1.0)
    mean_fg = (mu_total - mu) / jnp.where(weight_fg > 0.0, weight_fg, 1.0)
    icv = omega * weight_fg * (mean_bg - mean_fg) ** 2
    icv = jnp.where(valid, icv, -jnp.inf)
    m = jnp.max(icv, axis=1, keepdims=True)
    cand = jnp.where(icv == m, t, NBINS)
    best = jnp.min(cand, axis=1, keepdims=True)
    tidx = jnp.clip(best + 1, 0, NBINS - 1)
    mn = mn_ref[0, 0]
    mx = mx_ref[0, 0]
    s = tidx.astype(jnp.float32) / float(NBINS - 1)
    thr = mn * (1.0 - s) + mx * s
    return jnp.where(tidx == NBINS - 1, mx, thr)


def _mask_body(p_ref, mn_ref, mx_ref, x_ref, o_ref, thr_out_ref, thr_s):
    i = pl.program_id(0)
    j = pl.program_id(1)

    @pl.when((i == 0) & (j == 0))
    def _():
        thr = _compute_thresholds(p_ref, mn_ref, mx_ref)
        thr_s[...] = thr
        thr_out_ref[...] = thr

    row = i * 3 + j
    rid = lax.broadcasted_iota(jnp.int32, (NROWS, 1), 0)
    tval = jnp.sum(jnp.where(rid == row, thr_s[...], 0.0))
    x = x_ref[...]
    o_ref[...] = jnp.where(x <= tval, 0.0, x)


def _mask(x, partials, mn16, mx16):
    b, c, h, w = x.shape
    return pl.pallas_call(
        _mask_body,
        grid=(b, c),
        in_specs=[
            pl.BlockSpec(partials.shape, lambda i, j: (0,)),
            pl.BlockSpec((1, 16), lambda i, j: (0, 0)),
            pl.BlockSpec((1, 16), lambda i, j: (0, 0)),
            pl.BlockSpec((1, 1, h, w), lambda i, j: (i, j, 0, 0)),
        ],
        out_specs=[
            pl.BlockSpec((1, 1, h, w), lambda i, j: (i, j, 0, 0)),
            pl.BlockSpec((NROWS, 1), lambda i, j: (0, 0)),
        ],
        out_shape=[
            jax.ShapeDtypeStruct(x.shape, jnp.float32),
            jax.ShapeDtypeStruct((NROWS, 1), jnp.float32),
        ],
        scratch_shapes=[pltpu.VMEM((NROWS, 1), jnp.float32)],
    )(partials, mn16, mx16, x)


@jax.jit
def _run(x):
    mn16, mx16 = _minmax(x)
    partials = _hist_partials(
        x.reshape(NROWS, 512, 512), mn16.reshape(16), mx16.reshape(16)
    )
    out, thr = _mask(x, partials, mn16, mx16)
    return out, thr.reshape(NROWS)


def kernel(x, nbins):
    return _run(x)
